# baseline (speedup 1.0000x reference)
import functools

import jax
import jax.numpy as jnp
from jax import lax
from jax.experimental import pallas as pl
from jax.experimental.pallas import tpu as pltpu
from jax.experimental.pallas import tpu_sc as plsc

EMBED = 64
NC = 2
NS = 16
NW = NC * NS

CHUNK = 512


@functools.cache
def _make(B):
    assert B % (8 * NW) == 0
    b_per_w = B // NW
    assert b_per_w % CHUNK == 0
    nch = b_per_w // CHUNK
    mesh = plsc.VectorSubcoreMesh(core_axis_name="c", subcore_axis_name="s")

    @functools.partial(
        pl.kernel,
        mesh=mesh,
        out_type=jax.ShapeDtypeStruct((B, EMBED), jnp.float32),
        scratch_types=[
            pltpu.VMEM((b_per_w,), jnp.int32),
            pltpu.VMEM((CHUNK, EMBED), jnp.float32),
            pltpu.SemaphoreType.DMA,
        ],
        compiler_params=pltpu.CompilerParams(use_tc_tiling_on_sc=False),
    )
    def k(x_hbm, table_hbm, out_hbm, idx_v, rows_v, sem):
        wid = lax.axis_index("s") * NC + lax.axis_index("c")
        base = wid * b_per_w
        pltpu.sync_copy(x_hbm.at[pl.ds(base, b_per_w)], idx_v)

        def body(c, carry):
            off = c * CHUNK
            pltpu.async_copy(
                table_hbm.at[idx_v.at[pl.ds(off, CHUNK)]], rows_v, sem
            ).wait()
            pltpu.sync_copy(rows_v, out_hbm.at[pl.ds(base + off, CHUNK)])
            return carry

        lax.fori_loop(0, nch, body, 0)

    return k


def kernel(x, table):
    bsz, seq = x.shape
    flat = x.reshape(bsz * seq).astype(jnp.int32)
    out = _make(bsz * seq)(flat, table)
    return out.reshape(bsz, seq, EMBED)


# speedup vs baseline: 1.0222x; 1.0222x over previous
import functools

import jax
import jax.numpy as jnp
from jax import lax
from jax.experimental import pallas as pl
from jax.experimental.pallas import tpu as pltpu
from jax.experimental.pallas import tpu_sc as plsc

EMBED = 64
NC = 2
NS = 16
NW = NC * NS

CHUNK = 256
NBUF = 4


@functools.cache
def _make(B):
    assert B % (8 * NW) == 0
    b_per_w = B // NW
    assert b_per_w % (CHUNK * NBUF) == 0
    nch = b_per_w // CHUNK
    nsteps = nch // NBUF
    mesh = plsc.VectorSubcoreMesh(core_axis_name="c", subcore_axis_name="s")

    @functools.partial(
        pl.kernel,
        mesh=mesh,
        out_type=jax.ShapeDtypeStruct((B, EMBED), jnp.float32),
        scratch_types=[
            pltpu.VMEM((b_per_w,), jnp.int32),
            *[pltpu.VMEM((CHUNK, EMBED), jnp.float32) for _ in range(NBUF)],
            *[pltpu.SemaphoreType.DMA for _ in range(2 * NBUF)],
        ],
        compiler_params=pltpu.CompilerParams(use_tc_tiling_on_sc=False),
    )
    def k(x_hbm, table_hbm, out_hbm, idx_v, *bufs):
        rows = bufs[:NBUF]
        gsem = bufs[NBUF:2 * NBUF]
        psem = bufs[2 * NBUF:]
        wid = lax.axis_index("s") * NC + lax.axis_index("c")
        base = wid * b_per_w
        pltpu.sync_copy(x_hbm.at[pl.ds(base, b_per_w)], idx_v)

        def gather(g, b):
            pltpu.async_copy(
                table_hbm.at[idx_v.at[pl.ds(g * CHUNK, CHUNK)]], rows[b], gsem[b]
            )

        def wait_gather(g, b):
            pltpu.make_async_copy(
                table_hbm.at[idx_v.at[pl.ds(g * CHUNK, CHUNK)]], rows[b], gsem[b]
            ).wait()

        def put(g, b):
            pltpu.async_copy(
                rows[b], out_hbm.at[pl.ds(base + g * CHUNK, CHUNK)], psem[b]
            )

        def wait_put(g, b):
            pltpu.make_async_copy(
                rows[b], out_hbm.at[pl.ds(base + g * CHUNK, CHUNK)], psem[b]
            ).wait()

        for b in range(NBUF):
            gather(b, b)

        def body(s, carry):
            for b in range(NBUF):
                g = s * NBUF + b
                wait_gather(g, b)
                put(g, b)
                wait_put(g, b)
                gather(g + NBUF, b)
            return carry

        lax.fori_loop(0, nsteps - 1, body, 0)

        for b in range(NBUF):
            g = (nsteps - 1) * NBUF + b
            wait_gather(g, b)
            put(g, b)
            wait_put(g, b)

    return k


def kernel(x, table):
    bsz, seq = x.shape
    flat = x.reshape(bsz * seq).astype(jnp.int32)
    out = _make(bsz * seq)(flat, table)
    return out.reshape(bsz, seq, EMBED)


# speedup vs baseline: 1.0252x; 1.0029x over previous
import functools

import jax
import jax.numpy as jnp
from jax import lax
from jax.experimental import pallas as pl
from jax.experimental.pallas import tpu as pltpu
from jax.experimental.pallas import tpu_sc as plsc

EMBED = 64
NC = 2
NS = 16
NW = NC * NS

NBUF = 4


@functools.cache
def _make(BATCH, SEQ):
    assert BATCH % NW == 0
    r_per_w = BATCH // NW
    assert r_per_w % NBUF == 0
    nsteps = r_per_w // NBUF
    mesh = plsc.VectorSubcoreMesh(core_axis_name="c", subcore_axis_name="s")

    @functools.partial(
        pl.kernel,
        mesh=mesh,
        out_type=jax.ShapeDtypeStruct((BATCH, SEQ, EMBED), jnp.float32),
        scratch_types=[
            pltpu.VMEM((r_per_w, SEQ), jnp.int32),
            *[pltpu.VMEM((SEQ, EMBED), jnp.float32) for _ in range(NBUF)],
            *[pltpu.SemaphoreType.DMA for _ in range(2 * NBUF)],
        ],
        compiler_params=pltpu.CompilerParams(use_tc_tiling_on_sc=False),
    )
    def k(x_hbm, table_hbm, out_hbm, idx_v, *bufs):
        rows = bufs[:NBUF]
        gsem = bufs[NBUF:2 * NBUF]
        psem = bufs[2 * NBUF:]
        wid = lax.axis_index("s") * NC + lax.axis_index("c")
        base = wid * r_per_w
        pltpu.sync_copy(x_hbm.at[pl.ds(base, r_per_w), :], idx_v)

        def gather(g, b):
            pltpu.async_copy(table_hbm.at[idx_v.at[g]], rows[b], gsem[b])

        def wait_gather(g, b):
            pltpu.make_async_copy(
                table_hbm.at[idx_v.at[g]], rows[b], gsem[b]
            ).wait()

        def put(g, b):
            pltpu.async_copy(rows[b], out_hbm.at[base + g], psem[b])

        def wait_put(g, b):
            pltpu.make_async_copy(
                rows[b], out_hbm.at[base + g], psem[b]
            ).wait()

        for b in range(NBUF):
            gather(b, b)

        def body(s, carry):
            for b in range(NBUF):
                g = s * NBUF + b
                wait_gather(g, b)
                put(g, b)
                wait_put(g, b)
                gather(g + NBUF, b)
            return carry

        lax.fori_loop(0, nsteps - 1, body, 0)

        for b in range(NBUF):
            g = (nsteps - 1) * NBUF + b
            wait_gather(g, b)
            put(g, b)
            wait_put(g, b)

    return k


def kernel(x, table):
    bsz, seq = x.shape
    return _make(bsz, seq)(x.astype(jnp.int32), table)


# speedup vs baseline: 1.2523x; 1.2216x over previous
import functools

import jax
import jax.numpy as jnp
from jax import lax
from jax.experimental import pallas as pl
from jax.experimental.pallas import tpu as pltpu
from jax.experimental.pallas import tpu_sc as plsc

EMBED = 64
PADDED = 128
NC = 2
NS = 16
NW = NC * NS

NBUF = 4


@functools.cache
def _make(BATCH, SEQ):
    assert BATCH % NW == 0
    r_per_w = BATCH // NW
    n_idx = r_per_w * SEQ
    assert r_per_w % NBUF == 0
    nsteps = r_per_w // NBUF
    mesh = plsc.VectorSubcoreMesh(core_axis_name="c", subcore_axis_name="s")

    @functools.partial(
        pl.kernel,
        mesh=mesh,
        out_type=jax.ShapeDtypeStruct((BATCH, SEQ, PADDED), jnp.float32),
        scratch_types=[
            pltpu.VMEM((n_idx,), jnp.int32),
            *[pltpu.VMEM((SEQ, PADDED), jnp.float32) for _ in range(NBUF)],
            *[pltpu.SemaphoreType.DMA for _ in range(2 * NBUF)],
        ],
        compiler_params=pltpu.CompilerParams(use_tc_tiling_on_sc=True),
    )
    def k(x_hbm, table_hbm, out_hbm, idx_v, *bufs):
        rows = bufs[:NBUF]
        gsem = bufs[NBUF:2 * NBUF]
        psem = bufs[2 * NBUF:]
        wid = lax.axis_index("s") * NC + lax.axis_index("c")
        base = wid * r_per_w
        pltpu.sync_copy(x_hbm.at[pl.ds(base * SEQ, n_idx)], idx_v)

        def gather(g, b):
            pltpu.async_copy(
                table_hbm.at[idx_v.at[pl.ds(g * SEQ, SEQ)]], rows[b], gsem[b]
            )

        def wait_gather(g, b):
            pltpu.make_async_copy(
                table_hbm.at[idx_v.at[pl.ds(g * SEQ, SEQ)]], rows[b], gsem[b]
            ).wait()

        def put(g, b):
            pltpu.async_copy(rows[b], out_hbm.at[base + g], psem[b])

        def wait_put(g, b):
            pltpu.make_async_copy(
                rows[b], out_hbm.at[base + g], psem[b]
            ).wait()

        for b in range(NBUF):
            gather(b, b)

        def body(s, carry):
            for b in range(NBUF):
                g = s * NBUF + b
                wait_gather(g, b)
                put(g, b)
                wait_put(g, b)
                gather(g + NBUF, b)
            return carry

        lax.fori_loop(0, nsteps - 1, body, 0)

        for b in range(NBUF):
            g = (nsteps - 1) * NBUF + b
            wait_gather(g, b)
            put(g, b)
            wait_put(g, b)

    return k


def kernel(x, table):
    bsz, seq = x.shape
    x_flat = x.reshape(bsz * seq).astype(jnp.int32)
    table_p = jnp.pad(table, ((0, 0), (0, PADDED - EMBED)))
    out_p = _make(bsz, seq)(x_flat, table_p)
    return out_p[:, :, :EMBED]


# speedup vs baseline: 1.5368x; 1.2271x over previous
import functools

import jax
import jax.numpy as jnp
from jax import lax
from jax.experimental import pallas as pl
from jax.experimental.pallas import tpu as pltpu
from jax.experimental.pallas import tpu_sc as plsc

EMBED = 64
PADDED = 128
NC = 2
NS = 16
NW = NC * NS

NBUF = 4


@functools.cache
def _make(BATCH, SEQ):
    assert BATCH % NW == 0
    r_per_w = BATCH // NW
    n_idx = r_per_w * SEQ
    assert r_per_w % NBUF == 0
    nsteps = r_per_w // NBUF
    mesh = plsc.VectorSubcoreMesh(core_axis_name="c", subcore_axis_name="s")

    @functools.partial(
        pl.kernel,
        mesh=mesh,
        out_type=jax.ShapeDtypeStruct((BATCH, SEQ, PADDED), jnp.float32),
        scratch_types=[
            pltpu.VMEM((n_idx,), jnp.int32),
            *[pltpu.VMEM((SEQ, PADDED), jnp.float32) for _ in range(NBUF)],
            *[pltpu.SemaphoreType.DMA for _ in range(2 * NBUF)],
        ],
        compiler_params=pltpu.CompilerParams(use_tc_tiling_on_sc=True),
    )
    def k(x_hbm, table_hbm, out_hbm, idx_v, *bufs):
        rows = bufs[:NBUF]
        gsem = bufs[NBUF:2 * NBUF]
        psem = bufs[2 * NBUF:]
        wid = lax.axis_index("s") * NC + lax.axis_index("c")
        base = wid * r_per_w
        pltpu.sync_copy(x_hbm.at[pl.ds(base * SEQ, n_idx)], idx_v)

        def gather(g, b):
            pltpu.async_copy(
                table_hbm.at[idx_v.at[pl.ds(g * SEQ, SEQ)]], rows[b], gsem[b]
            )

        def wait_gather(g, b):
            pltpu.make_async_copy(
                table_hbm.at[idx_v.at[pl.ds(g * SEQ, SEQ)]], rows[b], gsem[b]
            ).wait()

        def put(g, b):
            pltpu.async_copy(rows[b], out_hbm.at[base + g], psem[b])

        def wait_put(g, b):
            pltpu.make_async_copy(
                rows[b], out_hbm.at[base + g], psem[b]
            ).wait()

        for b in range(NBUF):
            gather(b, b)

        def body(s, carry):
            for b in range(NBUF):
                g = s * NBUF + b
                wait_gather(g, b)
                put(g, b)
                wait_put(g, b)
                gather(g + NBUF, b)
            return carry

        lax.fori_loop(0, nsteps - 1, body, 0)

        for b in range(NBUF):
            g = (nsteps - 1) * NBUF + b
            wait_gather(g, b)
            put(g, b)
            wait_put(g, b)

    return k


TBLK = 4096


@functools.cache
def _make_transpose_pad(V):
    def body(tt_ref, out_ref):
        out_ref[:, :EMBED] = jnp.transpose(tt_ref[...])

    return pl.pallas_call(
        body,
        grid=((V + TBLK - 1) // TBLK,),
        in_specs=[pl.BlockSpec((EMBED, TBLK), lambda i: (0, i))],
        out_specs=pl.BlockSpec((TBLK, PADDED), lambda i: (i, 0)),
        out_shape=jax.ShapeDtypeStruct((V, PADDED), jnp.float32),
    )


def kernel(x, table):
    bsz, seq = x.shape
    x_flat = x.reshape(bsz * seq).astype(jnp.int32)
    table_p = _make_transpose_pad(table.shape[0])(table.T)
    out_p = _make(bsz, seq)(x_flat, table_p)
    return out_p[:, :, :EMBED]


# speedup vs baseline: 1.6767x; 1.0911x over previous
import functools

import jax
import jax.numpy as jnp
from jax import lax
from jax.experimental import pallas as pl
from jax.experimental.pallas import tpu as pltpu
from jax.experimental.pallas import tpu_sc as plsc

EMBED = 64
PADDED = 128
NC = 2
NS = 16
NW = NC * NS

NBUF = 4


@functools.cache
def _make(BATCH, SEQ):
    assert BATCH % NW == 0
    r_per_w = BATCH // NW
    n_idx = r_per_w * SEQ
    assert r_per_w % NBUF == 0
    nsteps = r_per_w // NBUF
    mesh = plsc.VectorSubcoreMesh(core_axis_name="c", subcore_axis_name="s")

    @functools.partial(
        pl.kernel,
        mesh=mesh,
        out_type=jax.ShapeDtypeStruct((BATCH, SEQ, PADDED), jnp.float32),
        scratch_types=[
            pltpu.VMEM((n_idx,), jnp.int32),
            *[pltpu.VMEM((SEQ, PADDED), jnp.float32) for _ in range(NBUF)],
            *[pltpu.SemaphoreType.DMA for _ in range(2 * NBUF)],
        ],
        compiler_params=pltpu.CompilerParams(use_tc_tiling_on_sc=True),
    )
    def k(x_hbm, table_hbm, out_hbm, idx_v, *bufs):
        rows = bufs[:NBUF]
        gsem = bufs[NBUF:2 * NBUF]
        psem = bufs[2 * NBUF:]
        wid = lax.axis_index("s") * NC + lax.axis_index("c")
        base = wid * r_per_w
        pltpu.sync_copy(x_hbm.at[pl.ds(base * SEQ, n_idx)], idx_v)

        def gather(g, b):
            pltpu.async_copy(
                table_hbm.at[idx_v.at[pl.ds(g * SEQ, SEQ)]], rows[b], gsem[b]
            )

        def wait_gather(g, b):
            pltpu.make_async_copy(
                table_hbm.at[idx_v.at[pl.ds(g * SEQ, SEQ)]], rows[b], gsem[b]
            ).wait()

        def put(g, b):
            pltpu.async_copy(rows[b], out_hbm.at[base + g], psem[b])

        def wait_put(g, b):
            pltpu.make_async_copy(
                rows[b], out_hbm.at[base + g], psem[b]
            ).wait()

        for b in range(NBUF):
            gather(b, b)

        def body(s, carry):
            for b in range(NBUF):
                g = s * NBUF + b
                wait_gather(g, b)
                put(g, b)
                wait_put(g, b)
                gather(g + NBUF, b)
            return carry

        lax.fori_loop(0, nsteps - 1, body, 0)

        for b in range(NBUF):
            g = (nsteps - 1) * NBUF + b
            wait_gather(g, b)
            put(g, b)
            wait_put(g, b)

    return k


TBLK = 8192


@functools.cache
def _make_transpose_pad(V):
    def body(tt_ref, out_ref):
        out_ref[:, :EMBED] = jnp.transpose(tt_ref[...])

    return pl.pallas_call(
        body,
        grid=((V + TBLK - 1) // TBLK,),
        in_specs=[pl.BlockSpec((EMBED, TBLK), lambda i: (0, i))],
        out_specs=pl.BlockSpec((TBLK, PADDED), lambda i: (i, 0)),
        out_shape=jax.ShapeDtypeStruct((V, PADDED), jnp.float32),
        compiler_params=pltpu.CompilerParams(
            dimension_semantics=("arbitrary",)
        ),
    )


def kernel(x, table):
    bsz, seq = x.shape
    x_flat = x.reshape(bsz * seq).astype(jnp.int32)
    table_p = _make_transpose_pad(table.shape[0])(table.T)
    out_p = _make(bsz, seq)(x_flat, table_p)
    return out_p[:, :, :EMBED]


# speedup vs baseline: 1.7234x; 1.0278x over previous
import functools

import jax
import jax.numpy as jnp
from jax import lax
from jax.experimental import pallas as pl
from jax.experimental.pallas import tpu as pltpu
from jax.experimental.pallas import tpu_sc as plsc

EMBED = 64
PADDED = 128
NC = 2
NS = 16
NW = NC * NS

NBUF = 4


@functools.cache
def _make(BATCH, SEQ):
    assert BATCH % NW == 0
    r_per_w = BATCH // NW
    n_idx = r_per_w * SEQ
    assert r_per_w % NBUF == 0
    nsteps = r_per_w // NBUF
    mesh = plsc.VectorSubcoreMesh(core_axis_name="c", subcore_axis_name="s")

    @functools.partial(
        pl.kernel,
        mesh=mesh,
        out_type=jax.ShapeDtypeStruct((BATCH, SEQ, PADDED), jnp.float32),
        scratch_types=[
            pltpu.VMEM((n_idx,), jnp.int32),
            *[pltpu.VMEM((SEQ, PADDED), jnp.float32) for _ in range(NBUF)],
            *[pltpu.SemaphoreType.DMA for _ in range(2 * NBUF)],
        ],
        compiler_params=pltpu.CompilerParams(use_tc_tiling_on_sc=True),
    )
    def k(x_hbm, table_hbm, out_hbm, idx_v, *bufs):
        rows = bufs[:NBUF]
        gsem = bufs[NBUF:2 * NBUF]
        psem = bufs[2 * NBUF:]
        wid = lax.axis_index("s") * NC + lax.axis_index("c")
        base = wid * r_per_w
        pltpu.sync_copy(x_hbm.at[pl.ds(base * SEQ, n_idx)], idx_v)

        def gather(g, b):
            pltpu.async_copy(
                table_hbm.at[idx_v.at[pl.ds(g * SEQ, SEQ)]], rows[b], gsem[b]
            )

        def wait_gather(g, b):
            pltpu.make_async_copy(
                table_hbm.at[idx_v.at[pl.ds(g * SEQ, SEQ)]], rows[b], gsem[b]
            ).wait()

        def put(g, b):
            pltpu.async_copy(rows[b], out_hbm.at[base + g], psem[b])

        def wait_put(g, b):
            pltpu.make_async_copy(
                rows[b], out_hbm.at[base + g], psem[b]
            ).wait()

        for b in range(NBUF):
            gather(b, b)

        def body(s, carry):
            for b in range(NBUF):
                g = s * NBUF + b
                wait_gather(g, b)
                put(g, b)
                wait_put(g, b)
                gather(g + NBUF, b)
            return carry

        lax.fori_loop(0, nsteps - 1, body, 0)

        for b in range(NBUF):
            g = (nsteps - 1) * NBUF + b
            wait_gather(g, b)
            put(g, b)
            wait_put(g, b)

    return k


TBLK = 16384


@functools.cache
def _make_transpose_pad(V):
    def body(tt_ref, out_ref):
        out_ref[:, :EMBED] = jnp.transpose(tt_ref[...])

    return pl.pallas_call(
        body,
        grid=((V + TBLK - 1) // TBLK,),
        in_specs=[pl.BlockSpec((EMBED, TBLK), lambda i: (0, i))],
        out_specs=pl.BlockSpec((TBLK, PADDED), lambda i: (i, 0)),
        out_shape=jax.ShapeDtypeStruct((V, PADDED), jnp.float32),
        compiler_params=pltpu.CompilerParams(
            dimension_semantics=("arbitrary",)
        ),
    )


def kernel(x, table):
    bsz, seq = x.shape
    x_flat = x.reshape(bsz * seq).astype(jnp.int32)
    table_p = _make_transpose_pad(table.shape[0])(table.T)
    out_p = _make(bsz, seq)(x_flat, table_p)
    return out_p[:, :, :EMBED]


# speedup vs baseline: 1.7409x; 1.0102x over previous
import functools

import jax
import jax.numpy as jnp
from jax import lax
from jax.experimental import pallas as pl
from jax.experimental.pallas import tpu as pltpu
from jax.experimental.pallas import tpu_sc as plsc

EMBED = 64
PADDED = 128
NC = 2
NS = 16
NW = NC * NS

NBUF = 4


@functools.cache
def _make(BATCH, SEQ):
    assert BATCH % NW == 0
    r_per_w = BATCH // NW
    n_idx = r_per_w * SEQ
    assert r_per_w % NBUF == 0
    nsteps = r_per_w // NBUF
    mesh = plsc.VectorSubcoreMesh(core_axis_name="c", subcore_axis_name="s")

    @functools.partial(
        pl.kernel,
        mesh=mesh,
        out_type=jax.ShapeDtypeStruct((BATCH, SEQ, PADDED), jnp.float32),
        scratch_types=[
            pltpu.VMEM((n_idx,), jnp.int32),
            *[pltpu.VMEM((SEQ, PADDED), jnp.float32) for _ in range(NBUF)],
            *[pltpu.SemaphoreType.DMA for _ in range(2 * NBUF)],
        ],
        compiler_params=pltpu.CompilerParams(use_tc_tiling_on_sc=True),
    )
    def k(x_hbm, table_hbm, out_hbm, idx_v, *bufs):
        rows = bufs[:NBUF]
        gsem = bufs[NBUF:2 * NBUF]
        psem = bufs[2 * NBUF:]
        wid = lax.axis_index("s") * NC + lax.axis_index("c")
        base = wid * r_per_w
        pltpu.sync_copy(x_hbm.at[pl.ds(base * SEQ, n_idx)], idx_v)

        def gather(g, b):
            pltpu.async_copy(
                table_hbm.at[idx_v.at[pl.ds(g * SEQ, SEQ)]], rows[b], gsem[b]
            )

        def wait_gather(g, b):
            pltpu.make_async_copy(
                table_hbm.at[idx_v.at[pl.ds(g * SEQ, SEQ)]], rows[b], gsem[b]
            ).wait()

        def put(g, b):
            pltpu.async_copy(rows[b], out_hbm.at[base + g], psem[b])

        def wait_put(g, b):
            pltpu.make_async_copy(
                rows[b], out_hbm.at[base + g], psem[b]
            ).wait()

        for b in range(NBUF):
            gather(b, b)

        def body(s, carry):
            for b in range(NBUF):
                g = s * NBUF + b
                wait_gather(g, b)
                put(g, b)
                wait_put(g, b)
                gather(g + NBUF, b)
            return carry

        lax.fori_loop(0, nsteps - 1, body, 0)

        for b in range(NBUF):
            g = (nsteps - 1) * NBUF + b
            wait_gather(g, b)
            put(g, b)
            wait_put(g, b)

    return k


TBLK = 32768


@functools.cache
def _make_transpose_pad(V):
    def body(tt_ref, out_ref):
        out_ref[:, :EMBED] = jnp.transpose(tt_ref[...])

    return pl.pallas_call(
        body,
        grid=((V + TBLK - 1) // TBLK,),
        in_specs=[pl.BlockSpec((EMBED, TBLK), lambda i: (0, i))],
        out_specs=pl.BlockSpec((TBLK, PADDED), lambda i: (i, 0)),
        out_shape=jax.ShapeDtypeStruct((V, PADDED), jnp.float32),
        compiler_params=pltpu.CompilerParams(
            dimension_semantics=("arbitrary",)
        ),
    )


def kernel(x, table):
    bsz, seq = x.shape
    x_flat = x.reshape(bsz * seq).astype(jnp.int32)
    table_p = _make_transpose_pad(table.shape[0])(table.T)
    out_p = _make(bsz, seq)(x_flat, table_p)
    return out_p[:, :, :EMBED]
